# baseline (speedup 1.0000x reference)
import functools

import jax
import jax.numpy as jnp
from jax import lax
from jax.experimental import pallas as pl
from jax.experimental.pallas import tpu as pltpu
import jax.experimental.pallas.tpu_sc as plsc

BATCH = 16384
D = 16
NC = 2
NS = 16
L = 16
NW = NC * NS
BPW = BATCH // NW
CHUNK = 128
NCHUNK = BPW // CHUNK


def _body(uid_hbm, iid_hbm, ut_hbm, it_hbm, w_hbm, b_hbm, out_hbm,
          idx_u, idx_i, u_rows, i_rows, out_v, wv, bv, sem):
  c = lax.axis_index("c")
  s = lax.axis_index("s")
  wid = s * NC + c
  base = wid * BPW

  pltpu.sync_copy(uid_hbm.at[pl.ds(base, BPW)], idx_u)
  pltpu.sync_copy(iid_hbm.at[pl.ds(base, BPW)], idx_i)
  pltpu.sync_copy(w_hbm, wv)
  pltpu.sync_copy(b_hbm, bv)

  cps = []
  for j in range(NCHUNK):
    sl = pl.ds(j * CHUNK, CHUNK)
    cps.append(pltpu.async_copy(ut_hbm.at[idx_u.at[sl]], u_rows.at[sl], sem))
    cps.append(pltpu.async_copy(it_hbm.at[idx_i.at[sl]], i_rows.at[sl], sem))
  for cp in cps:
    cp.wait()

  w_s = wv[...]
  b_s = bv[...]
  iota = lax.iota(jnp.int32, L)

  def block(g, _):
    row = g * L + iota
    acc = None
    for d in range(D):
      col = jnp.full((L,), d, jnp.int32)
      cu = plsc.load_gather(u_rows, [row, col])
      ci = plsc.load_gather(i_rows, [row, col])
      prod = cu * ci
      acc = prod if acc is None else acc + prod
    out_v[pl.ds(g * L, L)] = acc * w_s + b_s
    return 0

  lax.fori_loop(0, BPW // L, block, 0)

  pltpu.sync_copy(out_v, out_hbm.at[pl.ds(base, BPW)])


@jax.jit
def _run(userID, ItemID, user_table, item_table, w, b):
  mesh = plsc.VectorSubcoreMesh(core_axis_name="c", subcore_axis_name="s")
  f = pl.kernel(
      _body,
      out_type=jax.ShapeDtypeStruct((BATCH,), jnp.float32),
      mesh=mesh,
      scratch_types=[
          pltpu.VMEM((BPW,), jnp.int32),
          pltpu.VMEM((BPW,), jnp.int32),
          pltpu.VMEM((BPW, D), jnp.float32),
          pltpu.VMEM((BPW, D), jnp.float32),
          pltpu.VMEM((BPW,), jnp.float32),
          pltpu.VMEM((L,), jnp.float32),
          pltpu.VMEM((L,), jnp.float32),
          pltpu.SemaphoreType.DMA,
      ],
      compiler_params=pltpu.CompilerParams(needs_layout_passes=False, use_tc_tiling_on_sc=False),
  )
  return f(userID, ItemID, user_table, item_table, w, b)


def kernel(userID, ItemID, user_table, item_table, w, b):
  w16 = jnp.broadcast_to(jnp.reshape(w, (1,)), (L,))
  b16 = jnp.broadcast_to(jnp.reshape(b, (1,)), (L,))
  return _run(userID.astype(jnp.int32), ItemID.astype(jnp.int32),
              user_table, item_table, w16, b16)
